# baseline (speedup 1.0000x reference)
import jax
import jax.numpy as jnp
from jax.experimental import pallas as pl
from jax.experimental.pallas import tpu as pltpu

N = 10000
NFEAT = 128
NHID = 128
NCLASS = 16
SHID = N // 3

BM = 200
BA1 = 2000
BK1 = 1000
BK2 = 1024


def _a1_kernel(x_ref, wg_ref, xw_ref, s_ref):
    x = x_ref[...]
    xw_ref[...] = jax.lax.dot_general(
        x, wg_ref[...], (((1,), (0,)), ((), ())),
        preferred_element_type=jnp.float32)
    s_ref[...] = jnp.sum(x, axis=1, keepdims=True)


def _a2_kernel(s_ref, w1_ref, b1_ref, h_ref):
    i = pl.program_id(0)
    nsteps = pl.num_programs(0)
    part = jnp.sum(w1_ref[...] * s_ref[...], axis=0, keepdims=True)

    @pl.when(i == 0)
    def _():
        h_ref[...] = part

    @pl.when(i > 0)
    def _():
        h_ref[...] = h_ref[...] + part

    @pl.when(i == nsteps - 1)
    def _():
        h_ref[...] = jax.nn.relu(h_ref[...] + b1_ref[...])


def _a3_kernel(h_ref, w2_ref, b2_ref, sc_ref):
    acc = jax.lax.dot_general(
        h_ref[...], w2_ref[...], (((1,), (0,)), ((), ())),
        preferred_element_type=jnp.float32)
    sc_ref[...] = jax.nn.sigmoid(acc + b2_ref[...])


def _b_kernel(adj_ref, xw_ref, bg_ref, wcat_ref, hw_ref):
    h = jax.lax.dot_general(
        adj_ref[...], xw_ref[...], (((1,), (0,)), ((), ())),
        preferred_element_type=jnp.float32)
    h = jax.nn.relu(h + bg_ref[...])
    hw_ref[...] = jax.lax.dot_general(
        h, wcat_ref[...], (((1,), (0,)), ((), ())),
        preferred_element_type=jnp.float32)


def _c_kernel(adj_ref, hw_ref, sc_ref, eps_ref, b11_ref, b12_ref, out_ref,
              m_ref):
    i = pl.program_id(0)

    @pl.when(i == 0)
    def _():
        m_ref[...] = hw_ref[...] * sc_ref[...]

    acc = jax.lax.dot_general(
        adj_ref[...], m_ref[...], (((1,), (0,)), ((), ())),
        preferred_element_type=jnp.float32)
    mean = acc[:, :NCLASS] + b11_ref[...]
    logstd = acc[:, NCLASS:] + b12_ref[...]
    z = eps_ref[...] * jnp.exp(logstd) + mean
    zmax = jnp.max(z, axis=1, keepdims=True)
    ze = z - zmax
    out_ref[...] = ze - jnp.log(jnp.sum(jnp.exp(ze), axis=1, keepdims=True))


def kernel(x, adj, W_gc1, b_gc1, W_fc11, b_fc11, W_fc12, b_fc12,
           W_se1, b_se1, W_se2, b_se2, eps):
    f32 = jnp.float32

    xw, s = pl.pallas_call(
        _a1_kernel,
        grid=(N // BA1,),
        in_specs=[
            pl.BlockSpec((BA1, NFEAT), lambda i: (i, 0)),
            pl.BlockSpec((NFEAT, NHID), lambda i: (0, 0)),
        ],
        out_specs=[
            pl.BlockSpec((BA1, NHID), lambda i: (i, 0)),
            pl.BlockSpec((BA1, 1), lambda i: (i, 0)),
        ],
        out_shape=[
            jax.ShapeDtypeStruct((N, NHID), f32),
            jax.ShapeDtypeStruct((N, 1), f32),
        ],
        compiler_params=pltpu.CompilerParams(
            dimension_semantics=("parallel",)),
    )(x, W_gc1)

    h = pl.pallas_call(
        _a2_kernel,
        grid=(N // BK1,),
        in_specs=[
            pl.BlockSpec((BK1, 1), lambda i: (i, 0)),
            pl.BlockSpec((BK1, SHID), lambda i: (i, 0)),
            pl.BlockSpec((1, SHID), lambda i: (0, 0)),
        ],
        out_specs=pl.BlockSpec((1, SHID), lambda i: (0, 0)),
        out_shape=jax.ShapeDtypeStruct((1, SHID), f32),
        compiler_params=pltpu.CompilerParams(
            dimension_semantics=("arbitrary",)),
    )(s, W_se1, b_se1.reshape(1, SHID))

    sc_row = pl.pallas_call(
        _a3_kernel,
        grid=(pl.cdiv(N, BK2),),
        in_specs=[
            pl.BlockSpec((1, SHID), lambda i: (0, 0)),
            pl.BlockSpec((SHID, BK2), lambda i: (0, i)),
            pl.BlockSpec((1, BK2), lambda i: (0, i)),
        ],
        out_specs=pl.BlockSpec((1, BK2), lambda i: (0, i)),
        out_shape=jax.ShapeDtypeStruct((1, N), f32),
        compiler_params=pltpu.CompilerParams(
            dimension_semantics=("parallel",)),
    )(h, W_se2, b_se2.reshape(1, N))

    score = sc_row.reshape(N, 1)
    wcat = jnp.concatenate([W_fc11, W_fc12], axis=1)

    hw = pl.pallas_call(
        _b_kernel,
        grid=(N // BM,),
        in_specs=[
            pl.BlockSpec((BM, N), lambda i: (i, 0)),
            pl.BlockSpec((N, NHID), lambda i: (0, 0)),
            pl.BlockSpec((1, NHID), lambda i: (0, 0)),
            pl.BlockSpec((NHID, 2 * NCLASS), lambda i: (0, 0)),
        ],
        out_specs=pl.BlockSpec((BM, 2 * NCLASS), lambda i: (i, 0)),
        out_shape=jax.ShapeDtypeStruct((N, 2 * NCLASS), f32),
        compiler_params=pltpu.CompilerParams(
            dimension_semantics=("arbitrary",)),
    )(adj, xw, b_gc1.reshape(1, NHID), wcat)

    out = pl.pallas_call(
        _c_kernel,
        grid=(N // BM,),
        in_specs=[
            pl.BlockSpec((BM, N), lambda i: (i, 0)),
            pl.BlockSpec((N, 2 * NCLASS), lambda i: (0, 0)),
            pl.BlockSpec((N, 1), lambda i: (0, 0)),
            pl.BlockSpec((BM, NCLASS), lambda i: (i, 0)),
            pl.BlockSpec((1, NCLASS), lambda i: (0, 0)),
            pl.BlockSpec((1, NCLASS), lambda i: (0, 0)),
        ],
        out_specs=pl.BlockSpec((BM, NCLASS), lambda i: (i, 0)),
        out_shape=jax.ShapeDtypeStruct((N, NCLASS), f32),
        scratch_shapes=[pltpu.VMEM((N, 2 * NCLASS), f32)],
        compiler_params=pltpu.CompilerParams(
            dimension_semantics=("arbitrary",)),
    )(adj, hw, score, eps, b_fc11.reshape(1, NCLASS),
      b_fc12.reshape(1, NCLASS))

    return out
